# baseline (speedup 1.0000x reference)
import jax
import jax.numpy as jnp
from jax import lax
from jax.experimental import pallas as pl
from jax.experimental.pallas import tpu as pltpu
from jax.experimental.pallas import tpu_sc as plsc

EMB = 64
LANES = 16
SLAB = 128
TCBLK = 512


def _transpose_body(n_rows_padded, n_workers, n_cores):
    n_slabs = n_rows_padded // SLAB

    def body(src, out,
             in0, in1, tmp, ob0, ob1, isem0, isem1, osem0, osem1):
        wid = lax.axis_index("s") * n_cores + lax.axis_index("c")
        nk = (n_slabs - wid + n_workers - 1) // n_workers
        max_pairs = (n_slabs + n_workers - 1) // n_workers // 2 + 1

        iota = lax.iota(jnp.int32, LANES)
        jvecs = [j0 + iota for j0 in range(0, EMB, LANES)]
        pvecs = [((jv & 1) << 5) + (jv >> 1) for jv in jvecs]

        def fire_in(k, dst, sem):
            rt = wid + k * n_workers
            pltpu.async_copy(src.at[:, pl.ds(rt * SLAB, SLAB)], dst, sem)

        def wait_in(dst, sem):
            pltpu.make_async_copy(src.at[:, pl.ds(0, SLAB)], dst, sem).wait()

        def fire_out(k, sbuf, sem):
            rt = wid + k * n_workers
            pltpu.async_copy(sbuf, out.at[pl.ds(rt * SLAB, SLAB)], sem)

        def wait_out(sbuf, sem):
            pltpu.make_async_copy(sbuf, out.at[pl.ds(0, SLAB)], sem).wait()

        def transpose_slab(ibuf, obuf):
            def row_body(r, carry):
                for jvec, pvec in zip(jvecs, pvecs):
                    t = (jvec + r) & (SLAB - 1)
                    vals = plsc.load_gather(ibuf, [jvec, t])
                    plsc.store_scatter(tmp, [t, pvec], vals)
                return carry
            lax.fori_loop(0, SLAB, row_body, 0)

            def pack_body(r, carry):
                e0 = tmp[r, pl.ds(0, LANES)]
                e1 = tmp[r, pl.ds(LANES, LANES)]
                o0 = tmp[r, pl.ds(2 * LANES, LANES)]
                o1 = tmp[r, pl.ds(3 * LANES, LANES)]
                w0 = plsc.pack(e0, o0, format=plsc.PackFormat.INTERLEAVED)
                w1 = plsc.pack(e1, o1, format=plsc.PackFormat.INTERLEAVED)
                obuf[r, pl.ds(0, LANES)] = plsc.bitcast(w0, jnp.int32)
                obuf[r, pl.ds(LANES, LANES)] = plsc.bitcast(w1, jnp.int32)
                return carry
            lax.fori_loop(0, SLAB, pack_body, 0)

        fire_in(0, in0, isem0)
        fire_in(1, in1, isem1)

        def pair_body(p, carry):
            specs = ((2 * p, in0, ob0, isem0, osem0),
                     (2 * p + 1, in1, ob1, isem1, osem1))
            for k, ibuf, obuf, isem, osem in specs:
                @pl.when(k < nk)
                def _(k=k, ibuf=ibuf, obuf=obuf, isem=isem, osem=osem):
                    @pl.when(k >= 2)
                    def _():
                        wait_out(obuf, osem)
                    wait_in(ibuf, isem)
                    transpose_slab(ibuf, obuf)
                    fire_out(k, obuf, osem)

                    @pl.when(k + 2 < nk)
                    def _():
                        fire_in(k + 2, ibuf, isem)
            return carry

        lax.fori_loop(0, max_pairs, pair_body, 0)
        wait_out(ob0, osem0)
        wait_out(ob1, osem1)

    return body


def _tc_transpose_body(in_ref, out_ref):
    z = in_ref[...].reshape(EMB // 2, 2, TCBLK)
    et = jnp.transpose(z[:, 0, :], (1, 0))
    ot = jnp.transpose(z[:, 1, :], (1, 0))

    def rnd(x):
        b = lax.bitcast_convert_type(x, jnp.int32)
        return (b + 0x7FFF + ((b >> 16) & 1)) >> 16

    out_ref[...] = rnd(et) | (rnd(ot) << 16)


def _gather_body(bpw, n_cores):
    n_groups = bpw // LANES

    def body(uemb, iemb, uids, iids, out_hbm,
             uidx, iidx, ubuf, vbuf, outv, sem_a, sem_b):
        wid = lax.axis_index("s") * n_cores + lax.axis_index("c")
        base = wid * bpw

        pltpu.sync_copy(uids.at[pl.ds(base, bpw)], uidx)
        pltpu.sync_copy(iids.at[pl.ds(base, bpw)], iidx)

        lane = lax.iota(jnp.int32, LANES)

        def fire(g, slot, sem):
            uv = uidx[pl.ds(g * LANES, LANES)]
            iv = iidx[pl.ds(g * LANES, LANES)]
            for k in range(LANES):
                pltpu.async_copy(uemb.at[uv[k]], ubuf.at[slot, k], sem)
                pltpu.async_copy(iemb.at[iv[k]], vbuf.at[slot, k], sem)

        def drain(slot, sem):
            pltpu.make_async_copy(uemb.at[pl.ds(0, LANES)], ubuf.at[slot],
                                  sem).wait()
            pltpu.make_async_copy(iemb.at[pl.ds(0, LANES)], vbuf.at[slot],
                                  sem).wait()

        def halves(buf, slot, r):
            c0 = plsc.bitcast(buf[slot, r, pl.ds(0, LANES)], jnp.bfloat16)
            c1 = plsc.bitcast(buf[slot, r, pl.ds(LANES, LANES)], jnp.bfloat16)
            a0, b0 = plsc.unpack(c0, format=plsc.PackFormat.INTERLEAVED)
            a1, b1 = plsc.unpack(c1, format=plsc.PackFormat.INTERLEAVED)
            return a0, b0, a1, b1

        def compute(g, slot):
            out_vec = jnp.zeros((LANES,), jnp.float32)
            for r in range(LANES):
                u0, u1, u2, u3 = halves(ubuf, slot, r)
                v0, v1, v2, v3 = halves(vbuf, slot, r)
                acc = (u0 * v0 + u1 * v1) + (u2 * v2 + u3 * v3)
                s = jnp.sum(acc)
                out_vec = jnp.where(lane == r, s, out_vec)
            out_vec = 1.0 / (1.0 + jnp.exp(-out_vec))
            outv[pl.ds(g * LANES, LANES)] = out_vec

        fire(0, 0, sem_a)

        def pair_body(t, carry):
            g0 = 2 * t
            g1 = g0 + 1
            fire(g1, 1, sem_b)
            drain(0, sem_a)
            compute(g0, 0)

            @pl.when(t < (n_groups // 2) - 1)
            def _():
                fire(g0 + 2, 0, sem_a)

            drain(1, sem_b)
            compute(g1, 1)
            return carry

        lax.fori_loop(0, n_groups // 2, pair_body, 0)

        pltpu.sync_copy(outv, out_hbm.at[pl.ds(base, bpw)])

    return body


def kernel(user_emb, item_emb, user_ids, item_ids):
    B = user_ids.shape[0]
    n_items = item_emb.shape[0]
    info = plsc.get_sparse_core_info()
    n_cores, n_subcores = info.num_cores, info.num_subcores
    n_workers = n_cores * n_subcores
    bpw = B // n_workers
    n_rows_padded = (n_items + TCBLK - 1) // TCBLK * TCBLK

    mesh = plsc.VectorSubcoreMesh(core_axis_name="c", subcore_axis_name="s")
    params = pltpu.CompilerParams(
        needs_layout_passes=False, use_tc_tiling_on_sc=True,
        disable_bounds_checks=True)

    transpose = pl.kernel(
        _transpose_body(n_rows_padded, n_workers, n_cores),
        out_type=jax.ShapeDtypeStruct((n_rows_padded, EMB // 2), jnp.int32),
        mesh=mesh,
        compiler_params=params,
        scratch_types=[
            pltpu.VMEM((EMB, SLAB), jnp.float32),
            pltpu.VMEM((EMB, SLAB), jnp.float32),
            pltpu.VMEM((SLAB, EMB), jnp.float32),
            pltpu.VMEM((SLAB, EMB // 2), jnp.int32),
            pltpu.VMEM((SLAB, EMB // 2), jnp.int32),
            pltpu.SemaphoreType.DMA,
            pltpu.SemaphoreType.DMA,
            pltpu.SemaphoreType.DMA,
            pltpu.SemaphoreType.DMA,
        ],
    )

    tc_transpose = pl.pallas_call(
        _tc_transpose_body,
        grid=(n_rows_padded // TCBLK,),
        in_specs=[pl.BlockSpec((EMB, TCBLK), lambda i: (0, i))],
        out_specs=pl.BlockSpec((TCBLK, EMB // 2), lambda i: (i, 0)),
        out_shape=jax.ShapeDtypeStruct((n_rows_padded, EMB // 2), jnp.int32),
    )

    gather = pl.kernel(
        _gather_body(bpw, n_cores),
        out_type=jax.ShapeDtypeStruct((B,), jnp.float32),
        mesh=mesh,
        compiler_params=params,
        scratch_types=[
            pltpu.VMEM((bpw,), jnp.int32),
            pltpu.VMEM((bpw,), jnp.int32),
            pltpu.VMEM((2, LANES, EMB // 2), jnp.int32),
            pltpu.VMEM((2, LANES, EMB // 2), jnp.int32),
            pltpu.VMEM((bpw,), jnp.float32),
            pltpu.SemaphoreType.DMA,
            pltpu.SemaphoreType.DMA,
        ],
    )

    item_rm = transpose(item_emb.T)
    user_rm = tc_transpose(user_emb.T)
    return gather(user_rm, item_rm,
                  user_ids.astype(jnp.int32), item_ids.astype(jnp.int32))


# speedup vs baseline: 1.0103x; 1.0103x over previous
import jax
import jax.numpy as jnp
from jax import lax
from jax.experimental import pallas as pl
from jax.experimental.pallas import tpu as pltpu
from jax.experimental.pallas import tpu_sc as plsc

EMB = 64
LANES = 16
SLAB = 128
TCBLK = 512


def _transpose_body(n_rows_padded, n_workers, n_cores):
    n_slabs = n_rows_padded // SLAB

    def body(src, out,
             in0, in1, tmp, ob0, ob1, isem0, isem1, osem0, osem1):
        wid = lax.axis_index("s") * n_cores + lax.axis_index("c")
        nk = (n_slabs - wid + n_workers - 1) // n_workers
        max_pairs = (n_slabs + n_workers - 1) // n_workers // 2 + 1

        iota = lax.iota(jnp.int32, LANES)
        jvecs = [j0 + iota for j0 in range(0, EMB, LANES)]

        def fire_in(k, dst, sem):
            rt = wid + k * n_workers
            pltpu.async_copy(src.at[:, pl.ds(rt * SLAB, SLAB)], dst, sem)

        def wait_in(dst, sem):
            pltpu.make_async_copy(src.at[:, pl.ds(0, SLAB)], dst, sem).wait()

        def fire_out(k, sbuf, sem):
            rt = wid + k * n_workers
            pltpu.async_copy(sbuf, out.at[pl.ds(rt * SLAB, SLAB)], sem)

        def wait_out(sbuf, sem):
            pltpu.make_async_copy(sbuf, out.at[pl.ds(0, SLAB)], sem).wait()

        def transpose_slab(ibuf, obuf):
            def row_body(r, carry):
                for jvec in jvecs:
                    t = (jvec + r) & (SLAB - 1)
                    vals = plsc.load_gather(ibuf, [jvec, t])
                    plsc.store_scatter(tmp, [t, jvec], vals)
                return carry
            lax.fori_loop(0, SLAB, row_body, 0, unroll=8)

            def pack_body(r, carry):
                e0 = tmp[r, pl.ds(0, LANES)]
                e1 = tmp[r, pl.ds(LANES, LANES)]
                o0 = tmp[r, pl.ds(2 * LANES, LANES)]
                o1 = tmp[r, pl.ds(3 * LANES, LANES)]
                w0 = plsc.pack(e0, o0, format=plsc.PackFormat.INTERLEAVED)
                w1 = plsc.pack(e1, o1, format=plsc.PackFormat.INTERLEAVED)
                obuf[r, pl.ds(0, LANES)] = plsc.bitcast(w0, jnp.int32)
                obuf[r, pl.ds(LANES, LANES)] = plsc.bitcast(w1, jnp.int32)
                return carry
            lax.fori_loop(0, SLAB, pack_body, 0, unroll=8)

        fire_in(0, in0, isem0)
        fire_in(1, in1, isem1)

        def pair_body(p, carry):
            specs = ((2 * p, in0, ob0, isem0, osem0),
                     (2 * p + 1, in1, ob1, isem1, osem1))
            for k, ibuf, obuf, isem, osem in specs:
                @pl.when(k < nk)
                def _(k=k, ibuf=ibuf, obuf=obuf, isem=isem, osem=osem):
                    @pl.when(k >= 2)
                    def _():
                        wait_out(obuf, osem)
                    wait_in(ibuf, isem)
                    transpose_slab(ibuf, obuf)
                    fire_out(k, obuf, osem)

                    @pl.when(k + 2 < nk)
                    def _():
                        fire_in(k + 2, ibuf, isem)
            return carry

        lax.fori_loop(0, max_pairs, pair_body, 0)
        wait_out(ob0, osem0)
        wait_out(ob1, osem1)

    return body


def _tc_transpose_body(in_ref, out_ref):
    zt = jnp.transpose(in_ref[...], (1, 0))
    left = zt[:, 0:EMB // 2]
    right = zt[:, EMB // 2:EMB]

    def rnd(x):
        b = lax.bitcast_convert_type(x, jnp.int32)
        return (b + 0x7FFF + ((b >> 16) & 1)) >> 16

    out_ref[...] = rnd(left) | (rnd(right) << 16)


def _gather_body(bpw, n_cores):
    n_groups = bpw // LANES

    def body(uemb, iemb, uids, iids, out_hbm,
             uidx, iidx, ubuf, vbuf, outv, sem_a, sem_b):
        wid = lax.axis_index("s") * n_cores + lax.axis_index("c")
        base = wid * bpw

        pltpu.sync_copy(uids.at[pl.ds(base, bpw)], uidx)
        pltpu.sync_copy(iids.at[pl.ds(base, bpw)], iidx)

        lane = lax.iota(jnp.int32, LANES)

        def fire(g, slot, sem):
            uv = uidx[pl.ds(g * LANES, LANES)]
            iv = iidx[pl.ds(g * LANES, LANES)]
            for k in range(LANES):
                pltpu.async_copy(uemb.at[uv[k]], ubuf.at[slot, k], sem)
                pltpu.async_copy(iemb.at[iv[k]], vbuf.at[slot, k], sem)

        def drain(slot, sem):
            pltpu.make_async_copy(uemb.at[pl.ds(0, LANES)], ubuf.at[slot],
                                  sem).wait()
            pltpu.make_async_copy(iemb.at[pl.ds(0, LANES)], vbuf.at[slot],
                                  sem).wait()

        def halves(buf, slot, r):
            c0 = plsc.bitcast(buf[slot, r, pl.ds(0, LANES)], jnp.bfloat16)
            c1 = plsc.bitcast(buf[slot, r, pl.ds(LANES, LANES)], jnp.bfloat16)
            a0, b0 = plsc.unpack(c0, format=plsc.PackFormat.INTERLEAVED)
            a1, b1 = plsc.unpack(c1, format=plsc.PackFormat.INTERLEAVED)
            return a0, b0, a1, b1

        def compute(g, slot):
            out_vec = jnp.zeros((LANES,), jnp.float32)
            for r in range(LANES):
                u0, u1, u2, u3 = halves(ubuf, slot, r)
                v0, v1, v2, v3 = halves(vbuf, slot, r)
                acc = (u0 * v0 + u1 * v1) + (u2 * v2 + u3 * v3)
                s = jnp.sum(acc)
                out_vec = jnp.where(lane == r, s, out_vec)
            out_vec = 1.0 / (1.0 + jnp.exp(-out_vec))
            outv[pl.ds(g * LANES, LANES)] = out_vec

        fire(0, 0, sem_a)

        def pair_body(t, carry):
            g0 = 2 * t
            g1 = g0 + 1
            fire(g1, 1, sem_b)
            drain(0, sem_a)
            compute(g0, 0)

            @pl.when(t < (n_groups // 2) - 1)
            def _():
                fire(g0 + 2, 0, sem_a)

            drain(1, sem_b)
            compute(g1, 1)
            return carry

        lax.fori_loop(0, n_groups // 2, pair_body, 0)

        pltpu.sync_copy(outv, out_hbm.at[pl.ds(base, bpw)])

    return body


def kernel(user_emb, item_emb, user_ids, item_ids):
    B = user_ids.shape[0]
    n_items = item_emb.shape[0]
    info = plsc.get_sparse_core_info()
    n_cores, n_subcores = info.num_cores, info.num_subcores
    n_workers = n_cores * n_subcores
    bpw = B // n_workers
    n_rows_padded = (n_items + TCBLK - 1) // TCBLK * TCBLK

    mesh = plsc.VectorSubcoreMesh(core_axis_name="c", subcore_axis_name="s")
    params = pltpu.CompilerParams(
        needs_layout_passes=False, use_tc_tiling_on_sc=True,
        disable_bounds_checks=True)

    transpose = pl.kernel(
        _transpose_body(n_rows_padded, n_workers, n_cores),
        out_type=jax.ShapeDtypeStruct((n_rows_padded, EMB // 2), jnp.int32),
        mesh=mesh,
        compiler_params=params,
        scratch_types=[
            pltpu.VMEM((EMB, SLAB), jnp.float32),
            pltpu.VMEM((EMB, SLAB), jnp.float32),
            pltpu.VMEM((SLAB, EMB), jnp.float32),
            pltpu.VMEM((SLAB, EMB // 2), jnp.int32),
            pltpu.VMEM((SLAB, EMB // 2), jnp.int32),
            pltpu.SemaphoreType.DMA,
            pltpu.SemaphoreType.DMA,
            pltpu.SemaphoreType.DMA,
            pltpu.SemaphoreType.DMA,
        ],
    )

    tc_transpose = pl.pallas_call(
        _tc_transpose_body,
        grid=(n_rows_padded // TCBLK,),
        in_specs=[pl.BlockSpec((EMB, TCBLK), lambda i: (0, i))],
        out_specs=pl.BlockSpec((TCBLK, EMB // 2), lambda i: (i, 0)),
        out_shape=jax.ShapeDtypeStruct((n_rows_padded, EMB // 2), jnp.int32),
    )

    gather = pl.kernel(
        _gather_body(bpw, n_cores),
        out_type=jax.ShapeDtypeStruct((B,), jnp.float32),
        mesh=mesh,
        compiler_params=params,
        scratch_types=[
            pltpu.VMEM((bpw,), jnp.int32),
            pltpu.VMEM((bpw,), jnp.int32),
            pltpu.VMEM((2, LANES, EMB // 2), jnp.int32),
            pltpu.VMEM((2, LANES, EMB // 2), jnp.int32),
            pltpu.VMEM((bpw,), jnp.float32),
            pltpu.SemaphoreType.DMA,
            pltpu.SemaphoreType.DMA,
        ],
    )

    item_rm = transpose(item_emb.T)
    user_rm = tc_transpose(user_emb.T)
    return gather(user_rm, item_rm,
                  user_ids.astype(jnp.int32), item_ids.astype(jnp.int32))


# speedup vs baseline: 2.0325x; 2.0118x over previous
import jax
import jax.numpy as jnp
from jax import lax
from jax.experimental import pallas as pl
from jax.experimental.pallas import tpu as pltpu
from jax.experimental.pallas import tpu_sc as plsc

EMB = 64
LANES = 16
SLAB = 128
TCBLK = 512


def _transpose_body(n_rows_padded, n_workers, n_cores):
    n_slabs = n_rows_padded // SLAB

    def body(src, out,
             in0, in1, tmp, ob0, ob1, isem0, isem1, osem0, osem1):
        wid = lax.axis_index("s") * n_cores + lax.axis_index("c")
        nk = (n_slabs - wid + n_workers - 1) // n_workers
        max_pairs = (n_slabs + n_workers - 1) // n_workers // 2 + 1

        iota = lax.iota(jnp.int32, LANES)
        jvecs = [j0 + iota for j0 in range(0, EMB, LANES)]

        def fire_in(k, dst, sem):
            rt = wid + k * n_workers
            pltpu.async_copy(src.at[:, pl.ds(rt * SLAB, SLAB)], dst, sem)

        def wait_in(dst, sem):
            pltpu.make_async_copy(src.at[:, pl.ds(0, SLAB)], dst, sem).wait()

        def fire_out(k, sbuf, sem):
            rt = wid + k * n_workers
            pltpu.async_copy(sbuf, out.at[pl.ds(rt * SLAB, SLAB)], sem)

        def wait_out(sbuf, sem):
            pltpu.make_async_copy(sbuf, out.at[pl.ds(0, SLAB)], sem).wait()

        def transpose_slab(ibuf, obuf):
            def row_body(r, carry):
                for jvec in jvecs:
                    t = (jvec + r) & (SLAB - 1)
                    vals = plsc.load_gather(ibuf, [jvec, t])
                    plsc.store_scatter(tmp, [t, jvec], vals)
                return carry
            lax.fori_loop(0, SLAB, row_body, 0, unroll=8)

            def pack_body(r, carry):
                e0 = tmp[r, pl.ds(0, LANES)]
                e1 = tmp[r, pl.ds(LANES, LANES)]
                o0 = tmp[r, pl.ds(2 * LANES, LANES)]
                o1 = tmp[r, pl.ds(3 * LANES, LANES)]
                w0 = plsc.pack(e0, o0, format=plsc.PackFormat.INTERLEAVED)
                w1 = plsc.pack(e1, o1, format=plsc.PackFormat.INTERLEAVED)
                obuf[r, pl.ds(0, LANES)] = plsc.bitcast(w0, jnp.int32)
                obuf[r, pl.ds(LANES, LANES)] = plsc.bitcast(w1, jnp.int32)
                return carry
            lax.fori_loop(0, SLAB, pack_body, 0, unroll=8)

        fire_in(0, in0, isem0)
        fire_in(1, in1, isem1)

        def pair_body(p, carry):
            specs = ((2 * p, in0, ob0, isem0, osem0),
                     (2 * p + 1, in1, ob1, isem1, osem1))
            for k, ibuf, obuf, isem, osem in specs:
                @pl.when(k < nk)
                def _(k=k, ibuf=ibuf, obuf=obuf, isem=isem, osem=osem):
                    @pl.when(k >= 2)
                    def _():
                        wait_out(obuf, osem)
                    wait_in(ibuf, isem)
                    transpose_slab(ibuf, obuf)
                    fire_out(k, obuf, osem)

                    @pl.when(k + 2 < nk)
                    def _():
                        fire_in(k + 2, ibuf, isem)
            return carry

        lax.fori_loop(0, max_pairs, pair_body, 0)
        wait_out(ob0, osem0)
        wait_out(ob1, osem1)

    return body


def _tc_transpose_body(in_ref, out_ref):
    zt = jnp.transpose(in_ref[...], (1, 0))
    left = zt[:, 0:EMB // 2]
    right = zt[:, EMB // 2:EMB]

    def rnd(x):
        b = lax.bitcast_convert_type(x, jnp.int32)
        return (b + 0x7FFF + ((b >> 16) & 1)) >> 16

    out_ref[...] = rnd(left) | (rnd(right) << 16)


def _gather_body(bpw, n_cores):
    n_groups = bpw // LANES

    def body(uemb, iemb, uids, iids, out_hbm,
             uidx, iidx, ubuf, vbuf, outv, sem_a, sem_b):
        wid = lax.axis_index("s") * n_cores + lax.axis_index("c")
        base = wid * bpw

        pltpu.sync_copy(uids.at[pl.ds(base, bpw)], uidx)
        pltpu.sync_copy(iids.at[pl.ds(base, bpw)], iidx)

        lane = lax.iota(jnp.int32, LANES)

        def fire(g, slot, sem):
            uv = uidx[pl.ds(g * LANES, LANES)]
            iv = iidx[pl.ds(g * LANES, LANES)]
            for k in range(LANES):
                pltpu.async_copy(uemb.at[uv[k]], ubuf.at[slot, k], sem)
                pltpu.async_copy(iemb.at[iv[k]], vbuf.at[slot, k], sem)

        def uhalves(buf, slot, r):
            u0 = buf[slot, r, pl.ds(0, LANES)]
            u1 = buf[slot, r, pl.ds(LANES, LANES)]
            u2 = buf[slot, r, pl.ds(2 * LANES, LANES)]
            u3 = buf[slot, r, pl.ds(3 * LANES, LANES)]
            return u0, u2, u1, u3

        def drain(slot, sem):
            pltpu.make_async_copy(uemb.at[pl.ds(0, LANES)], ubuf.at[slot],
                                  sem).wait()
            pltpu.make_async_copy(iemb.at[pl.ds(0, LANES)], vbuf.at[slot],
                                  sem).wait()

        def halves(buf, slot, r):
            c0 = plsc.bitcast(buf[slot, r, pl.ds(0, LANES)], jnp.bfloat16)
            c1 = plsc.bitcast(buf[slot, r, pl.ds(LANES, LANES)], jnp.bfloat16)
            a0, b0 = plsc.unpack(c0, format=plsc.PackFormat.INTERLEAVED)
            a1, b1 = plsc.unpack(c1, format=plsc.PackFormat.INTERLEAVED)
            return a0, b0, a1, b1

        def compute(g, slot):
            out_vec = jnp.zeros((LANES,), jnp.float32)
            for r in range(LANES):
                u0, u1, u2, u3 = uhalves(ubuf, slot, r)
                v0, v1, v2, v3 = halves(vbuf, slot, r)
                acc = (u0 * v0 + u1 * v1) + (u2 * v2 + u3 * v3)
                s = jnp.sum(acc)
                out_vec = jnp.where(lane == r, s, out_vec)
            out_vec = 1.0 / (1.0 + jnp.exp(-out_vec))
            outv[pl.ds(g * LANES, LANES)] = out_vec

        fire(0, 0, sem_a)

        def pair_body(t, carry):
            g0 = 2 * t
            g1 = g0 + 1
            fire(g1, 1, sem_b)
            drain(0, sem_a)
            compute(g0, 0)

            @pl.when(t < (n_groups // 2) - 1)
            def _():
                fire(g0 + 2, 0, sem_a)

            drain(1, sem_b)
            compute(g1, 1)
            return carry

        lax.fori_loop(0, n_groups // 2, pair_body, 0)

        pltpu.sync_copy(outv, out_hbm.at[pl.ds(base, bpw)])

    return body


def kernel(user_emb, item_emb, user_ids, item_ids):
    B = user_ids.shape[0]
    n_items = item_emb.shape[0]
    info = plsc.get_sparse_core_info()
    n_cores, n_subcores = info.num_cores, info.num_subcores
    n_workers = n_cores * n_subcores
    bpw = B // n_workers
    n_rows_padded = (n_items + TCBLK - 1) // TCBLK * TCBLK

    mesh = plsc.VectorSubcoreMesh(core_axis_name="c", subcore_axis_name="s")
    params = pltpu.CompilerParams(
        needs_layout_passes=False, use_tc_tiling_on_sc=True,
        disable_bounds_checks=True)

    transpose = pl.kernel(
        _transpose_body(n_rows_padded, n_workers, n_cores),
        out_type=jax.ShapeDtypeStruct((n_rows_padded, EMB // 2), jnp.int32),
        mesh=mesh,
        compiler_params=params,
        scratch_types=[
            pltpu.VMEM((EMB, SLAB), jnp.float32),
            pltpu.VMEM((EMB, SLAB), jnp.float32),
            pltpu.VMEM((SLAB, EMB), jnp.float32),
            pltpu.VMEM((SLAB, EMB // 2), jnp.int32),
            pltpu.VMEM((SLAB, EMB // 2), jnp.int32),
            pltpu.SemaphoreType.DMA,
            pltpu.SemaphoreType.DMA,
            pltpu.SemaphoreType.DMA,
            pltpu.SemaphoreType.DMA,
        ],
    )

    gather = pl.kernel(
        _gather_body(bpw, n_cores),
        out_type=jax.ShapeDtypeStruct((B,), jnp.float32),
        mesh=mesh,
        compiler_params=params,
        scratch_types=[
            pltpu.VMEM((bpw,), jnp.int32),
            pltpu.VMEM((bpw,), jnp.int32),
            pltpu.VMEM((2, LANES, EMB), jnp.float32),
            pltpu.VMEM((2, LANES, EMB // 2), jnp.int32),
            pltpu.VMEM((bpw,), jnp.float32),
            pltpu.SemaphoreType.DMA,
            pltpu.SemaphoreType.DMA,
        ],
    )

    item_rm = transpose(item_emb.T)
    return gather(user_emb, item_rm,
                  user_ids.astype(jnp.int32), item_ids.astype(jnp.int32))


# speedup vs baseline: 2.0572x; 1.0121x over previous
import jax
import jax.numpy as jnp
from jax import lax
from jax.experimental import pallas as pl
from jax.experimental.pallas import tpu as pltpu
from jax.experimental.pallas import tpu_sc as plsc

EMB = 64
LANES = 16


def _gather_body(bpw, n_cores):
    n_groups = bpw // LANES

    def body(uemb, iemb, uids, iids, out_hbm,
             uidx, iidx, ubuf, vbuf, outv, sem_a, sem_b):
        wid = lax.axis_index("s") * n_cores + lax.axis_index("c")
        base = wid * bpw

        pltpu.sync_copy(uids.at[pl.ds(base, bpw)], uidx)
        pltpu.sync_copy(iids.at[pl.ds(base, bpw)], iidx)

        lane = lax.iota(jnp.int32, LANES)

        def fire(g, slot, sem):
            uv = uidx[pl.ds(g * LANES, LANES)]
            iv = iidx[pl.ds(g * LANES, LANES)]
            for k in range(LANES):
                pltpu.async_copy(uemb.at[uv[k]], ubuf.at[slot, k], sem)
                pltpu.async_copy(iemb.at[iv[k]], vbuf.at[slot, k], sem)

        def drain(slot, sem):
            pltpu.make_async_copy(uemb.at[pl.ds(0, LANES)], ubuf.at[slot],
                                  sem).wait()
            pltpu.make_async_copy(iemb.at[pl.ds(0, LANES)], vbuf.at[slot],
                                  sem).wait()

        def quarters(buf, slot, r):
            return (buf[slot, r, pl.ds(0, LANES)],
                    buf[slot, r, pl.ds(LANES, LANES)],
                    buf[slot, r, pl.ds(2 * LANES, LANES)],
                    buf[slot, r, pl.ds(3 * LANES, LANES)])

        def compute(g, slot):
            out_vec = jnp.zeros((LANES,), jnp.float32)
            for r in range(LANES):
                u0, u1, u2, u3 = quarters(ubuf, slot, r)
                v0, v1, v2, v3 = quarters(vbuf, slot, r)
                acc = (u0 * v0 + u1 * v1) + (u2 * v2 + u3 * v3)
                s = jnp.sum(acc)
                out_vec = jnp.where(lane == r, s, out_vec)
            out_vec = 1.0 / (1.0 + jnp.exp(-out_vec))
            outv[pl.ds(g * LANES, LANES)] = out_vec

        fire(0, 0, sem_a)

        def pair_body(t, carry):
            g0 = 2 * t
            g1 = g0 + 1
            fire(g1, 1, sem_b)
            drain(0, sem_a)
            compute(g0, 0)

            @pl.when(t < (n_groups // 2) - 1)
            def _():
                fire(g0 + 2, 0, sem_a)

            drain(1, sem_b)
            compute(g1, 1)
            return carry

        lax.fori_loop(0, n_groups // 2, pair_body, 0)

        pltpu.sync_copy(outv, out_hbm.at[pl.ds(base, bpw)])

    return body


def kernel(user_emb, item_emb, user_ids, item_ids):
    B = user_ids.shape[0]
    info = plsc.get_sparse_core_info()
    n_cores, n_subcores = info.num_cores, info.num_subcores
    n_workers = n_cores * n_subcores
    bpw = B // n_workers

    mesh = plsc.VectorSubcoreMesh(core_axis_name="c", subcore_axis_name="s")
    params = pltpu.CompilerParams(
        needs_layout_passes=False, use_tc_tiling_on_sc=True,
        disable_bounds_checks=True)

    gather = pl.kernel(
        _gather_body(bpw, n_cores),
        out_type=jax.ShapeDtypeStruct((B,), jnp.float32),
        mesh=mesh,
        compiler_params=params,
        scratch_types=[
            pltpu.VMEM((bpw,), jnp.int32),
            pltpu.VMEM((bpw,), jnp.int32),
            pltpu.VMEM((2, LANES, EMB), jnp.float32),
            pltpu.VMEM((2, LANES, EMB), jnp.float32),
            pltpu.VMEM((bpw,), jnp.float32),
            pltpu.SemaphoreType.DMA,
            pltpu.SemaphoreType.DMA,
        ],
    )

    return gather(user_emb, item_emb,
                  user_ids.astype(jnp.int32), item_ids.astype(jnp.int32))


# speedup vs baseline: 2.0791x; 1.0107x over previous
import jax
import jax.numpy as jnp
from jax import lax
from jax.experimental import pallas as pl
from jax.experimental.pallas import tpu as pltpu
from jax.experimental.pallas import tpu_sc as plsc

EMB = 64
LANES = 16


def _gather_body(bpw, n_cores):
    n_groups = bpw // LANES

    def body(uemb, iemb, uids, iids, out_hbm,
             uidx, iidx, ubuf, vbuf, outv, sem_a, sem_b, sem_c, sem_d):
        wid = lax.axis_index("s") * n_cores + lax.axis_index("c")
        base = wid * bpw

        pltpu.sync_copy(uids.at[pl.ds(base, bpw)], uidx)
        pltpu.sync_copy(iids.at[pl.ds(base, bpw)], iidx)

        lane = lax.iota(jnp.int32, LANES)

        def fire(g, slot, sem):
            uv = uidx[pl.ds(g * LANES, LANES)]
            iv = iidx[pl.ds(g * LANES, LANES)]
            for k in range(LANES):
                pltpu.async_copy(uemb.at[uv[k]], ubuf.at[slot, k], sem)
                pltpu.async_copy(iemb.at[iv[k]], vbuf.at[slot, k], sem)

        def drain(slot, sem):
            pltpu.make_async_copy(uemb.at[pl.ds(0, LANES)], ubuf.at[slot],
                                  sem).wait()
            pltpu.make_async_copy(iemb.at[pl.ds(0, LANES)], vbuf.at[slot],
                                  sem).wait()

        def quarters(buf, slot, r):
            return (buf[slot, r, pl.ds(0, LANES)],
                    buf[slot, r, pl.ds(LANES, LANES)],
                    buf[slot, r, pl.ds(2 * LANES, LANES)],
                    buf[slot, r, pl.ds(3 * LANES, LANES)])

        def compute(g, slot):
            out_vec = jnp.zeros((LANES,), jnp.float32)
            for r in range(LANES):
                u0, u1, u2, u3 = quarters(ubuf, slot, r)
                v0, v1, v2, v3 = quarters(vbuf, slot, r)
                acc = (u0 * v0 + u1 * v1) + (u2 * v2 + u3 * v3)
                s = jnp.sum(acc)
                out_vec = jnp.where(lane == r, s, out_vec)
            out_vec = 1.0 / (1.0 + jnp.exp(-out_vec))
            outv[pl.ds(g * LANES, LANES)] = out_vec

        sems = (sem_a, sem_b, sem_c, sem_d)
        ns = len(sems)
        for s in range(ns - 1):
            fire(s, s, sems[s])

        def quad_body(t, carry):
            g0 = ns * t
            for j in range(ns):
                prev = g0 + j - 1 + ns

                @pl.when(prev < n_groups)
                def _(prev=prev, j=j):
                    fire(prev, (j - 1) % ns, sems[(j - 1) % ns])

                drain(j, sems[j])
                compute(g0 + j, j)
            return carry

        lax.fori_loop(0, n_groups // ns, quad_body, 0)

        pltpu.sync_copy(outv, out_hbm.at[pl.ds(base, bpw)])

    return body


def kernel(user_emb, item_emb, user_ids, item_ids):
    B = user_ids.shape[0]
    info = plsc.get_sparse_core_info()
    n_cores, n_subcores = info.num_cores, info.num_subcores
    n_workers = n_cores * n_subcores
    bpw = B // n_workers

    mesh = plsc.VectorSubcoreMesh(core_axis_name="c", subcore_axis_name="s")
    params = pltpu.CompilerParams(
        needs_layout_passes=False, use_tc_tiling_on_sc=True,
        disable_bounds_checks=True)

    gather = pl.kernel(
        _gather_body(bpw, n_cores),
        out_type=jax.ShapeDtypeStruct((B,), jnp.float32),
        mesh=mesh,
        compiler_params=params,
        scratch_types=[
            pltpu.VMEM((bpw,), jnp.int32),
            pltpu.VMEM((bpw,), jnp.int32),
            pltpu.VMEM((4, LANES, EMB), jnp.float32),
            pltpu.VMEM((4, LANES, EMB), jnp.float32),
            pltpu.VMEM((bpw,), jnp.float32),
            pltpu.SemaphoreType.DMA,
            pltpu.SemaphoreType.DMA,
            pltpu.SemaphoreType.DMA,
            pltpu.SemaphoreType.DMA,
        ],
    )

    return gather(user_emb, item_emb,
                  user_ids.astype(jnp.int32), item_ids.astype(jnp.int32))
